# baseline (speedup 1.0000x reference)
import functools

import jax
import jax.numpy as jnp
from jax import lax
from jax.experimental import pallas as pl
from jax.experimental.pallas import tpu as pltpu
from jax.experimental.pallas import tpu_sc as plsc

EMBED_DIM = 2048
NUM_EXPERTS = 16
TOP_K = 2

BLOCK_T = 2048
NSPLIT = 8
SUB_T = BLOCK_T // NSPLIT

LANES = 16
N_WORKERS = 32


def _matmul_block(x_hbm, w_ref, logits_ref, x_buf, sems):
    i = pl.program_id(0)
    nsteps = pl.num_programs(0)

    def copy(step, slot, s):
        return pltpu.make_async_copy(
            x_hbm.at[pl.ds(step * BLOCK_T + s * SUB_T, SUB_T), :],
            x_buf.at[slot, pl.ds(s * SUB_T, SUB_T), :],
            sems.at[slot, s],
        )

    slot = lax.rem(i, 2)
    nxt = lax.rem(i + 1, 2)

    @pl.when(i == 0)
    def _first():
        for s in range(NSPLIT):
            copy(0, 0, s).start()

    @pl.when(i + 1 < nsteps)
    def _prefetch():
        for s in range(NSPLIT):
            copy(i + 1, nxt, s).start()

    for s in range(NSPLIT):
        copy(i, slot, s).wait()

    logits_ref[...] = jax.lax.dot_general(
        x_buf[slot], w_ref[...],
        dimension_numbers=(((1,), (1,)), ((), ())),
        preferred_element_type=jnp.float32,
    )


def _tc_logits(x, W):
    n_tokens = x.shape[0]
    return pl.pallas_call(
        _matmul_block,
        grid=(n_tokens // BLOCK_T,),
        in_specs=[
            pl.BlockSpec(memory_space=pl.ANY),
            pl.BlockSpec((NUM_EXPERTS, EMBED_DIM), lambda i: (0, 0)),
        ],
        out_specs=pl.BlockSpec((BLOCK_T, NUM_EXPERTS), lambda i: (i, 0)),
        out_shape=jax.ShapeDtypeStruct((n_tokens, NUM_EXPERTS), jnp.float32),
        scratch_shapes=[
            pltpu.VMEM((2, BLOCK_T, EMBED_DIM), jnp.float32),
            pltpu.SemaphoreType.DMA((2, NSPLIT)),
        ],
    )(x, W)


def _make_sc_top2(n_tokens):
    tok_per_w = n_tokens // N_WORKERS
    groups = tok_per_w // LANES
    mesh = plsc.VectorSubcoreMesh(core_axis_name="c", subcore_axis_name="s")

    @functools.partial(
        pl.kernel, mesh=mesh,
        compiler_params=pltpu.CompilerParams(needs_layout_passes=False),
        out_type=(
            jax.ShapeDtypeStruct((n_tokens * TOP_K,), jnp.int32),
            jax.ShapeDtypeStruct((n_tokens * TOP_K,), jnp.float32),
        ),
        scratch_types=[
            pltpu.VMEM((tok_per_w * NUM_EXPERTS,), jnp.float32),
            pltpu.VMEM((tok_per_w * TOP_K,), jnp.int32),
            pltpu.VMEM((tok_per_w * TOP_K,), jnp.float32),
        ],
    )
    def sc_top2(logits_hbm, idx_hbm, wgt_hbm, lg_v, idx_v, wgt_v):
        c = lax.axis_index("c")
        s = lax.axis_index("s")
        wid = s * 2 + c
        base = wid * tok_per_w

        pltpu.sync_copy(
            logits_hbm.at[pl.ds(base * NUM_EXPERTS, tok_per_w * NUM_EXPERTS)],
            lg_v)

        lane = lax.broadcasted_iota(jnp.int32, (LANES,), 0)
        lane16 = lane * NUM_EXPERTS
        lane2 = lane * TOP_K

        def group_body(g, _):
            gbase = g * (LANES * NUM_EXPERTS)
            e = [
                plsc.load_gather(lg_v, [lane16 + (gbase + j)])
                for j in range(NUM_EXPERTS)
            ]
            m1 = e[0]
            i1 = jnp.zeros((LANES,), jnp.int32)
            for j in range(1, NUM_EXPERTS):
                cond = e[j] > m1
                m1 = jnp.where(cond, e[j], m1)
                i1 = jnp.where(cond, j, i1)
            m2 = jnp.full((LANES,), -jnp.inf, jnp.float32)
            i2 = jnp.zeros((LANES,), jnp.int32)
            for j in range(NUM_EXPERTS):
                cond = jnp.logical_and(e[j] > m2, i1 != j)
                m2 = jnp.where(cond, e[j], m2)
                i2 = jnp.where(cond, j, i2)
            w1 = 1.0 / (1.0 + jnp.exp(m2 - m1))
            loc = lane2 + g * (LANES * TOP_K)
            plsc.store_scatter(idx_v, [loc], i1)
            plsc.store_scatter(idx_v, [loc + 1], i2)
            plsc.store_scatter(wgt_v, [loc], w1)
            plsc.store_scatter(wgt_v, [loc + 1], 1.0 - w1)
            return 0

        lax.fori_loop(0, groups, group_body, 0)

        pltpu.sync_copy(idx_v, idx_hbm.at[pl.ds(base * TOP_K,
                                                tok_per_w * TOP_K)])
        pltpu.sync_copy(wgt_v, wgt_hbm.at[pl.ds(base * TOP_K,
                                                tok_per_w * TOP_K)])

    return sc_top2


def kernel(x, W):
    n_tokens = x.shape[0]
    logits = _tc_logits(x, W)
    idx_flat, wgt_flat = _make_sc_top2(n_tokens)(logits.reshape(-1))
    idx = idx_flat.reshape(n_tokens, TOP_K)
    wgt = wgt_flat.reshape(n_tokens, TOP_K)
    return (idx, wgt, logits)


# speedup vs baseline: 1.0617x; 1.0617x over previous
import functools

import jax
import jax.numpy as jnp
from jax import lax
from jax.experimental import pallas as pl
from jax.experimental.pallas import tpu as pltpu
from jax.experimental.pallas import tpu_sc as plsc

EMBED_DIM = 2048
NUM_EXPERTS = 16
TOP_K = 2

BLOCK_T = 2048
NSPLIT = 8
SUB_T = BLOCK_T // NSPLIT

LANES = 16
N_WORKERS = 32


def _matmul_block(x_hbm, w_ref, logits_ref, x_buf, sems):
    i = pl.program_id(0)
    nsteps = pl.num_programs(0)

    def copy(step, slot, s):
        return pltpu.make_async_copy(
            x_hbm.at[pl.ds(step * BLOCK_T + s * SUB_T, SUB_T), :],
            x_buf.at[slot, pl.ds(s * SUB_T, SUB_T), :],
            sems.at[slot, s],
        )

    slot = lax.rem(i, 2)
    nxt = lax.rem(i + 1, 2)

    @pl.when(i == 0)
    def _first():
        for s in range(NSPLIT):
            copy(0, 0, s).start()

    @pl.when(i + 1 < nsteps)
    def _prefetch():
        for s in range(NSPLIT):
            copy(i + 1, nxt, s).start()

    for s in range(NSPLIT):
        copy(i, slot, s).wait()

    logits_ref[...] = jax.lax.dot_general(
        x_buf[slot], w_ref[...],
        dimension_numbers=(((1,), (1,)), ((), ())),
        preferred_element_type=jnp.float32,
    )


def _tc_logits(x, W):
    n_tokens = x.shape[0]
    return pl.pallas_call(
        _matmul_block,
        grid=(n_tokens // BLOCK_T,),
        in_specs=[
            pl.BlockSpec(memory_space=pl.ANY),
            pl.BlockSpec((NUM_EXPERTS, EMBED_DIM), lambda i: (0, 0)),
        ],
        out_specs=pl.BlockSpec((BLOCK_T, NUM_EXPERTS), lambda i: (i, 0)),
        out_shape=jax.ShapeDtypeStruct((n_tokens, NUM_EXPERTS), jnp.float32),
        scratch_shapes=[
            pltpu.VMEM((2, BLOCK_T, EMBED_DIM), jnp.float32),
            pltpu.SemaphoreType.DMA((2, NSPLIT)),
        ],
    )(x, W)


def _make_sc_top2(n_tokens):
    tok_per_w = n_tokens // N_WORKERS
    groups = tok_per_w // LANES
    mesh = plsc.VectorSubcoreMesh(core_axis_name="c", subcore_axis_name="s")

    @functools.partial(
        pl.kernel, mesh=mesh,
        compiler_params=pltpu.CompilerParams(
            needs_layout_passes=False, use_tc_tiling_on_sc=False),
        out_type=(
            jax.ShapeDtypeStruct((n_tokens, TOP_K), jnp.int32),
            jax.ShapeDtypeStruct((n_tokens, TOP_K), jnp.float32),
        ),
        scratch_types=[
            pltpu.VMEM((tok_per_w, NUM_EXPERTS), jnp.float32),
            pltpu.VMEM((tok_per_w, TOP_K), jnp.int32),
            pltpu.VMEM((tok_per_w, TOP_K), jnp.float32),
        ],
    )
    def sc_top2(logits_hbm, idx_hbm, wgt_hbm, lg_v, idx_v, wgt_v):
        c = lax.axis_index("c")
        s = lax.axis_index("s")
        wid = s * 2 + c
        base = wid * tok_per_w

        pltpu.sync_copy(logits_hbm.at[pl.ds(base, tok_per_w)], lg_v)

        lane = lax.broadcasted_iota(jnp.int32, (LANES,), 0)
        zero_i = jnp.zeros((LANES,), jnp.int32)

        def group_body(g, _):
            tok = lane + g * LANES
            e = [
                plsc.load_gather(
                    lg_v, [tok, jnp.full((LANES,), j, jnp.int32)])
                for j in range(NUM_EXPERTS)
            ]
            m1 = e[0]
            i1 = jnp.zeros((LANES,), jnp.int32)
            for j in range(1, NUM_EXPERTS):
                cond = e[j] > m1
                m1 = jnp.where(cond, e[j], m1)
                i1 = jnp.where(cond, j, i1)
            m2 = jnp.full((LANES,), -jnp.inf, jnp.float32)
            i2 = jnp.zeros((LANES,), jnp.int32)
            for j in range(NUM_EXPERTS):
                cond = jnp.logical_and(e[j] > m2, i1 != j)
                m2 = jnp.where(cond, e[j], m2)
                i2 = jnp.where(cond, j, i2)
            w1 = 1.0 / (1.0 + jnp.exp(m2 - m1))
            plsc.store_scatter(idx_v, [tok, zero_i], i1)
            plsc.store_scatter(idx_v, [tok, zero_i + 1], i2)
            plsc.store_scatter(wgt_v, [tok, zero_i], w1)
            plsc.store_scatter(wgt_v, [tok, zero_i + 1], 1.0 - w1)
            return 0

        lax.fori_loop(0, groups, group_body, 0)

        pltpu.sync_copy(idx_v, idx_hbm.at[pl.ds(base, tok_per_w)])
        pltpu.sync_copy(wgt_v, wgt_hbm.at[pl.ds(base, tok_per_w)])

    return sc_top2


def kernel(x, W):
    n_tokens = x.shape[0]
    logits = _tc_logits(x, W)
    idx, wgt = _make_sc_top2(n_tokens)(logits)
    return (idx, wgt, logits)


# speedup vs baseline: 2.3418x; 2.2058x over previous
import jax
import jax.numpy as jnp
from jax import lax
from jax.experimental import pallas as pl
from jax.experimental.pallas import tpu as pltpu

EMBED_DIM = 2048
NUM_EXPERTS = 16
TOP_K = 2

BLOCK_T = 2048
NSPLIT = 8
SUB_T = BLOCK_T // NSPLIT


def _router_block(x_hbm, w_ref, lt_ref, pk_ref, pw_ref, x_buf, sems):
    i = pl.program_id(0)
    nsteps = pl.num_programs(0)

    def copy(step, slot, s):
        return pltpu.make_async_copy(
            x_hbm.at[pl.ds(step * BLOCK_T + s * SUB_T, SUB_T), :],
            x_buf.at[slot, pl.ds(s * SUB_T, SUB_T), :],
            sems.at[slot, s],
        )

    slot = lax.rem(i, 2)
    nxt = lax.rem(i + 1, 2)

    @pl.when(i == 0)
    def _first():
        for s in range(NSPLIT):
            copy(0, 0, s).start()

    @pl.when(i + 1 < nsteps)
    def _prefetch():
        for s in range(NSPLIT):
            copy(i + 1, nxt, s).start()

    for s in range(NSPLIT):
        copy(i, slot, s).wait()

    lt = jax.lax.dot_general(
        w_ref[...], x_buf[slot],
        dimension_numbers=(((1,), (1,)), ((), ())),
        preferred_element_type=jnp.float32,
    )
    lt_ref[...] = lt

    iota = lax.broadcasted_iota(jnp.int32, lt.shape, 0)
    m1 = jnp.max(lt, axis=0, keepdims=True)
    i1 = jnp.min(jnp.where(lt == m1, iota, NUM_EXPERTS), axis=0,
                 keepdims=True)
    masked = jnp.where(iota == i1, -jnp.inf, lt)
    m2 = jnp.max(masked, axis=0, keepdims=True)
    i2 = jnp.min(jnp.where(masked == m2, iota, NUM_EXPERTS), axis=0,
                 keepdims=True)
    w1 = jax.nn.sigmoid(m1 - m2)
    pk_ref[...] = jnp.concatenate([i1, i2], axis=0)
    pw_ref[...] = jnp.concatenate([w1, 1.0 - w1], axis=0)


def kernel(x, W):
    n_tokens = x.shape[0]
    lt, pk, pw = pl.pallas_call(
        _router_block,
        grid=(n_tokens // BLOCK_T,),
        in_specs=[
            pl.BlockSpec(memory_space=pl.ANY),
            pl.BlockSpec((NUM_EXPERTS, EMBED_DIM), lambda i: (0, 0)),
        ],
        out_specs=(
            pl.BlockSpec((NUM_EXPERTS, BLOCK_T), lambda i: (0, i)),
            pl.BlockSpec((TOP_K, BLOCK_T), lambda i: (0, i)),
            pl.BlockSpec((TOP_K, BLOCK_T), lambda i: (0, i)),
        ),
        out_shape=(
            jax.ShapeDtypeStruct((NUM_EXPERTS, n_tokens), jnp.float32),
            jax.ShapeDtypeStruct((TOP_K, n_tokens), jnp.int32),
            jax.ShapeDtypeStruct((TOP_K, n_tokens), jnp.float32),
        ),
        scratch_shapes=[
            pltpu.VMEM((2, BLOCK_T, EMBED_DIM), jnp.float32),
            pltpu.SemaphoreType.DMA((2, NSPLIT)),
        ],
    )(x, W)
    return (pk.T, pw.T, lt.T)


# speedup vs baseline: 2.4400x; 1.0419x over previous
import jax
import jax.numpy as jnp
from jax import lax
from jax.experimental import pallas as pl
from jax.experimental.pallas import tpu as pltpu

EMBED_DIM = 2048
NUM_EXPERTS = 16
TOP_K = 2

BLOCK_T = 1024
NSPLIT = 8
SUB_T = BLOCK_T // NSPLIT


def _router_block(x_hbm, w_ref, lt_ref, pk_ref, pw_ref, x_buf, sems):
    i = pl.program_id(0)
    nsteps = pl.num_programs(0)

    def copy(step, slot, s):
        return pltpu.make_async_copy(
            x_hbm.at[pl.ds(step * BLOCK_T + s * SUB_T, SUB_T), :],
            x_buf.at[slot, pl.ds(s * SUB_T, SUB_T), :],
            sems.at[slot, s],
        )

    slot = lax.rem(i, 2)
    nxt = lax.rem(i + 1, 2)

    @pl.when(i == 0)
    def _first():
        for s in range(NSPLIT):
            copy(0, 0, s).start()

    @pl.when(i + 1 < nsteps)
    def _prefetch():
        for s in range(NSPLIT):
            copy(i + 1, nxt, s).start()

    for s in range(NSPLIT):
        copy(i, slot, s).wait()

    lt = jax.lax.dot_general(
        w_ref[...], x_buf[slot],
        dimension_numbers=(((1,), (1,)), ((), ())),
        preferred_element_type=jnp.float32,
    )
    lt_ref[...] = lt

    iota = lax.broadcasted_iota(jnp.int32, lt.shape, 0)
    m1 = jnp.max(lt, axis=0, keepdims=True)
    i1 = jnp.min(jnp.where(lt == m1, iota, NUM_EXPERTS), axis=0,
                 keepdims=True)
    masked = jnp.where(iota == i1, -jnp.inf, lt)
    m2 = jnp.max(masked, axis=0, keepdims=True)
    i2 = jnp.min(jnp.where(masked == m2, iota, NUM_EXPERTS), axis=0,
                 keepdims=True)
    w1 = jax.nn.sigmoid(m1 - m2)
    pk_ref[...] = jnp.concatenate([i1, i2], axis=0)
    pw_ref[...] = jnp.concatenate([w1, 1.0 - w1], axis=0)


def kernel(x, W):
    n_tokens = x.shape[0]
    lt, pk, pw = pl.pallas_call(
        _router_block,
        grid=(n_tokens // BLOCK_T,),
        in_specs=[
            pl.BlockSpec(memory_space=pl.ANY),
            pl.BlockSpec((NUM_EXPERTS, EMBED_DIM), lambda i: (0, 0)),
        ],
        out_specs=(
            pl.BlockSpec((NUM_EXPERTS, BLOCK_T), lambda i: (0, i)),
            pl.BlockSpec((TOP_K, BLOCK_T), lambda i: (0, i)),
            pl.BlockSpec((TOP_K, BLOCK_T), lambda i: (0, i)),
        ),
        out_shape=(
            jax.ShapeDtypeStruct((NUM_EXPERTS, n_tokens), jnp.float32),
            jax.ShapeDtypeStruct((TOP_K, n_tokens), jnp.int32),
            jax.ShapeDtypeStruct((TOP_K, n_tokens), jnp.float32),
        ),
        scratch_shapes=[
            pltpu.VMEM((2, BLOCK_T, EMBED_DIM), jnp.float32),
            pltpu.SemaphoreType.DMA((2, NSPLIT)),
        ],
    )(x, W)
    return (pk.T, pw.T, lt.T)
